# Initial kernel scaffold; baseline (speedup 1.0000x reference)
import jax
import jax.numpy as jnp
from jax.experimental import pallas as pl


def kernel(x, sample_xs, W_edge, b_edge, W_gcn, b_gcn, W1, b1, W2, b2, W3, b3):
    raise NotImplementedError("write your pallas kernel here")


# speedup vs baseline: 62.2695x; 62.2695x over previous
import jax
import jax.numpy as jnp
import numpy as np
from jax.experimental import pallas as pl
from jax.experimental.pallas import tpu as pltpu

N = 100000
D = 64
BLK = 5000
GRID = N // BLK

_INV_SQRT2 = np.float32(1.0 / np.sqrt(2.0))
_HALF = np.float32(0.5)


def _fused_body(x_ref, xs_ref, we_ref, be_ref, wg_ref, bg_ref,
                w1_ref, b1_ref, w2_ref, b2_ref, w3_ref, b3_ref,
                out_ref,
                wc_ref, bc_ref, xt_ref, hq_ref, m_ref, z_ref, w_ref):
    i = pl.program_id(0)

    @pl.when(i == 0)
    def _init():
        we = we_ref[...]
        wg = wg_ref[...]
        wc_ref[:, :D] = we
        wc_ref[:, D:] = jnp.dot(we, wg, preferred_element_type=jnp.float32)
        be = be_ref[...]
        bc_ref[:, :D] = be
        bc_ref[:, D:] = jnp.dot(be, wg, preferred_element_type=jnp.float32)
        q = jnp.dot(x_ref[...], wc_ref[...],
                    preferred_element_type=jnp.float32) + bc_ref[...]
        xt_ref[...] = q[:, :D]
        hq_ref[...] = q[:, D:]
        m_ref[...] = jnp.full((1, D), -jnp.inf, jnp.float32)
        z_ref[...] = jnp.zeros((1, D), jnp.float32)
        w_ref[...] = jnp.zeros((1, D), jnp.float32)

    sh = jnp.dot(xs_ref[...], wc_ref[...],
                 preferred_element_type=jnp.float32) + bc_ref[...]
    s = sh[:, :D]
    h = sh[:, D:]
    t = s * xt_ref[...]
    m_old = m_ref[...]
    m_new = jnp.maximum(m_old, jnp.max(t, axis=0, keepdims=True))
    alpha = jnp.exp(m_old - m_new)
    p = jnp.exp(t - m_new)
    z_ref[...] = z_ref[...] * alpha + jnp.sum(p, axis=0, keepdims=True)
    w_ref[...] = w_ref[...] * alpha + jnp.sum(p * h, axis=0, keepdims=True)
    m_ref[...] = m_new

    @pl.when(i == GRID - 1)
    def _fin():
        r = w_ref[...] / z_ref[...]
        agg = _INV_SQRT2 * r + _HALF * hq_ref[...] + bg_ref[...]
        a = jnp.where(agg > 0, agg, jnp.exp(jnp.minimum(agg, 0.0)) - 1.0)
        h1 = jnp.maximum(jnp.dot(a, w1_ref[...],
                                 preferred_element_type=jnp.float32)
                         + b1_ref[...], 0.0)
        h2 = jnp.maximum(jnp.dot(h1, w2_ref[...],
                                 preferred_element_type=jnp.float32)
                         + b2_ref[...], 0.0)
        h3 = jnp.maximum(jnp.dot(h2, w3_ref[...],
                                 preferred_element_type=jnp.float32)
                         + b3_ref[...], 0.0)
        e = jnp.exp(h3 - jnp.max(h3, axis=1, keepdims=True))
        out_ref[...] = e / jnp.sum(e, axis=1, keepdims=True)


def kernel(x, sample_xs, W_edge, b_edge, W_gcn, b_gcn, W1, b1, W2, b2, W3, b3):
    out = pl.pallas_call(
        _fused_body,
        grid=(GRID,),
        in_specs=[
            pl.BlockSpec((1, D), lambda i: (0, 0)),
            pl.BlockSpec((BLK, D), lambda i: (i, 0)),
            pl.BlockSpec((D, D), lambda i: (0, 0)),
            pl.BlockSpec((1, D), lambda i: (0, 0)),
            pl.BlockSpec((D, D), lambda i: (0, 0)),
            pl.BlockSpec((1, D), lambda i: (0, 0)),
            pl.BlockSpec((D, 128), lambda i: (0, 0)),
            pl.BlockSpec((1, 128), lambda i: (0, 0)),
            pl.BlockSpec((128, 16), lambda i: (0, 0)),
            pl.BlockSpec((1, 16), lambda i: (0, 0)),
            pl.BlockSpec((16, 16), lambda i: (0, 0)),
            pl.BlockSpec((1, 16), lambda i: (0, 0)),
        ],
        out_specs=pl.BlockSpec((1, 16), lambda i: (0, 0)),
        out_shape=jax.ShapeDtypeStruct((1, 16), jnp.float32),
        scratch_shapes=[
            pltpu.VMEM((D, 2 * D), jnp.float32),
            pltpu.VMEM((1, 2 * D), jnp.float32),
            pltpu.VMEM((1, D), jnp.float32),
            pltpu.VMEM((1, D), jnp.float32),
            pltpu.VMEM((1, D), jnp.float32),
            pltpu.VMEM((1, D), jnp.float32),
            pltpu.VMEM((1, D), jnp.float32),
        ],
        compiler_params=pltpu.CompilerParams(
            dimension_semantics=("arbitrary",)),
    )(x, sample_xs, W_edge, b_edge.reshape(1, D), W_gcn, b_gcn.reshape(1, D),
      W1, b1.reshape(1, 128), W2, b2.reshape(1, 16), W3, b3.reshape(1, 16))
    return out.reshape(16)
